# Initial kernel scaffold; baseline (speedup 1.0000x reference)
import jax
import jax.numpy as jnp
from jax.experimental import pallas as pl


def kernel(inputs, emb_tok, W, b, pos_emb):
    raise NotImplementedError("write your pallas kernel here")


# speedup vs baseline: 16.4736x; 16.4736x over previous
import functools

import jax
import jax.numpy as jnp
from jax import lax
from jax.experimental import pallas as pl
from jax.experimental.pallas import tpu as pltpu
from jax.experimental.pallas import tpu_sc as plsc

B, S, C = 1024, 10, 25
V, D, H = 64, 400, 16
P = S * C
N = B * S * C
NC, NS, L = 2, 16, 16
NW = NC * NS
PER_W = N // NW
GB = 80
CH = 2000
NG = CH // GB
NCHUNK = PER_W // CH


def _table_body(emb_tok_ref, w_ref, b_ref, posc_ref, out_ref):
    proj = jnp.dot(emb_tok_ref[...], w_ref[...],
                   preferred_element_type=jnp.float32)
    proj = proj + b_ref[...]
    out_ref[...] = posc_ref[...][:, None, :] + proj[None, :, :]


def _build_table(emb_tok, W, b, posc):
    return pl.pallas_call(
        _table_body,
        out_shape=jax.ShapeDtypeStruct((P, V, H), jnp.float32),
    )(emb_tok, W, b.reshape(1, H), posc)


def _sc_body(idx_hbm, table_hbm, out_hbm, idxc_v, idx2_v, rows_v, sem):
    wid = lax.axis_index("s") * NC + lax.axis_index("c")
    wbase = wid * PER_W
    lane = lax.iota(jnp.int32, L)

    def chunk_body(k, carry):
        cbase = pl.multiple_of(wbase + k * CH, CH)
        pltpu.sync_copy(idx_hbm.at[pl.ds(cbase, CH)], idxc_v)
        for g in range(NG):
            for j in range(GB // L):
                o = g * GB + j * L
                n0 = cbase + o
                p = (n0 + lane) % P
                idx2_v[g, pl.ds(j * L, L)] = p * V + idxc_v[pl.ds(o, L)]
        copies = []
        for g in range(NG):
            copies.append(pltpu.async_copy(
                table_hbm.at[idx2_v.at[g]],
                rows_v.at[pl.ds(g * GB, GB)], sem))
        for c in copies:
            c.wait()
        pltpu.sync_copy(rows_v, out_hbm.at[pl.ds(cbase, CH)])
        return carry

    lax.fori_loop(0, NCHUNK, chunk_body, 0)


@functools.partial(
    pl.kernel,
    out_type=jax.ShapeDtypeStruct((N, H), jnp.float32),
    mesh=plsc.VectorSubcoreMesh(core_axis_name="c", subcore_axis_name="s"),
    scratch_types=[
        pltpu.VMEM((CH,), jnp.int32),
        pltpu.VMEM((NG, GB), jnp.int32),
        pltpu.VMEM((CH, H), jnp.float32),
        pltpu.SemaphoreType.DMA,
    ],
    compiler_params=pltpu.CompilerParams(use_tc_tiling_on_sc=False),
)
def _sc_lookup(idx_hbm, table_hbm, out_hbm, idxc_v, idx2_v, rows_v, sem):
    _sc_body(idx_hbm, table_hbm, out_hbm, idxc_v, idx2_v, rows_v, sem)


def kernel(inputs, emb_tok, W, b, pos_emb):
    posc = pos_emb.reshape(P, H)
    combined = _build_table(emb_tok, W, b, posc).reshape(P * V, H)
    idx_flat = inputs.reshape(N).astype(jnp.int32)
    out_flat = _sc_lookup(idx_flat, combined)
    return out_flat.reshape(B, S, D)
